# baseline (speedup 1.0000x reference)
import functools

import jax
import jax.numpy as jnp
from jax import lax
from jax.experimental import pallas as pl
from jax.experimental.pallas import tpu as pltpu
from jax.experimental.pallas import tpu_sc as plsc

NC, NS = 2, 16
NW = NC * NS
CHUNK = 128


def _embed_call(n_chunks, features, x_parts, embedding):
    mesh = plsc.VectorSubcoreMesh(core_axis_name="c", subcore_axis_name="s")

    @functools.partial(
        pl.kernel,
        out_type=jax.ShapeDtypeStruct((NW, n_chunks, CHUNK, features),
                                      jnp.float32),
        mesh=mesh,
        scratch_types=[
            pltpu.VMEM((n_chunks, CHUNK), jnp.int32),
            pltpu.VMEM((CHUNK, features), jnp.float32),
            pltpu.SemaphoreType.DMA,
        ],
        compiler_params=pltpu.CompilerParams(use_tc_tiling_on_sc=False),
    )
    def embed(x_hbm, table_hbm, out_hbm, idx_v, rows_v, sem):
        wid = lax.axis_index("s") * NC + lax.axis_index("c")
        pltpu.sync_copy(x_hbm.at[wid], idx_v)

        def chunk_body(j, carry):
            pltpu.async_copy(table_hbm.at[idx_v.at[j]], rows_v, sem).wait()
            pltpu.sync_copy(rows_v, out_hbm.at[wid, j])
            return carry

        lax.fori_loop(0, n_chunks, chunk_body, 0)

    return embed(x_parts, embedding)


def kernel(x, embedding):
    B, S = x.shape
    V, D = embedding.shape
    total = B * S
    per_w = total // NW
    n_chunks = per_w // CHUNK
    x_parts = x.reshape(NW, n_chunks, CHUNK).astype(jnp.int32)
    out = _embed_call(n_chunks, D, x_parts, embedding)
    return out.reshape(B, S, D)


# speedup vs baseline: 1.0491x; 1.0491x over previous
import functools

import jax
import jax.numpy as jnp
from jax import lax
from jax.experimental import pallas as pl
from jax.experimental.pallas import tpu as pltpu
from jax.experimental.pallas import tpu_sc as plsc

NC, NS = 2, 16
NW = NC * NS
CHUNK = 128
NBUF = 8


def _embed_call(n_chunks, features, x_parts, embedding):
    mesh = plsc.VectorSubcoreMesh(core_axis_name="c", subcore_axis_name="s")

    @functools.partial(
        pl.kernel,
        out_type=jax.ShapeDtypeStruct((NW, n_chunks, CHUNK, features),
                                      jnp.float32),
        mesh=mesh,
        scratch_types=[
            pltpu.VMEM((n_chunks, CHUNK), jnp.int32),
            pltpu.VMEM((NBUF, CHUNK, features), jnp.float32),
            pltpu.SemaphoreType.DMA,
            pltpu.SemaphoreType.DMA,
        ],
        compiler_params=pltpu.CompilerParams(use_tc_tiling_on_sc=False),
    )
    def embed(x_hbm, table_hbm, out_hbm, idx_v, rows_v, sem_g, sem_s):
        wid = lax.axis_index("s") * NC + lax.axis_index("c")
        pltpu.sync_copy(x_hbm.at[wid], idx_v)

        for b in range(NBUF):
            pltpu.async_copy(table_hbm.at[idx_v.at[b]], rows_v.at[b], sem_g)

        def wait_one_gather():
            pltpu.make_async_copy(
                table_hbm.at[pl.ds(0, CHUNK)], rows_v.at[0], sem_g).wait()

        def wait_one_store():
            pltpu.make_async_copy(
                rows_v.at[0], out_hbm.at[0, 0], sem_s).wait()

        def chunk_body(j, carry):
            b = lax.rem(j, NBUF)
            wait_one_gather()
            pltpu.async_copy(rows_v.at[b], out_hbm.at[wid, j], sem_s)

            @pl.when(j + NBUF < n_chunks)
            def _refill():
                wait_one_store()
                pltpu.async_copy(
                    table_hbm.at[idx_v.at[j + NBUF]], rows_v.at[b], sem_g)

            return carry

        lax.fori_loop(0, n_chunks, chunk_body, 0)

        for _ in range(NBUF):
            wait_one_store()

    return embed(x_parts, embedding)


def kernel(x, embedding):
    B, S = x.shape
    V, D = embedding.shape
    total = B * S
    per_w = total // NW
    n_chunks = per_w // CHUNK
    x_parts = x.reshape(NW, n_chunks, CHUNK).astype(jnp.int32)
    out = _embed_call(n_chunks, D, x_parts, embedding)
    return out.reshape(B, S, D)


# speedup vs baseline: 1.1544x; 1.1004x over previous
import functools

import jax
import jax.numpy as jnp
from jax import lax
from jax.experimental import pallas as pl
from jax.experimental.pallas import tpu as pltpu
from jax.experimental.pallas import tpu_sc as plsc

NC, NS = 2, 16
NW = NC * NS
LB = 128
GROUP = 4


def _embed_call(S, V4, x_t, table4):
    mesh = plsc.VectorSubcoreMesh(core_axis_name="c", subcore_axis_name="s")

    @functools.partial(
        pl.kernel,
        out_type=jax.ShapeDtypeStruct((S, 4, NW, 8, LB), jnp.float32),
        mesh=mesh,
        scratch_types=[
            pltpu.VMEM((S, LB), jnp.int32),
            pltpu.VMEM((2, LB), jnp.int32),
            pltpu.VMEM((2, LB), jnp.int32),
            pltpu.VMEM((2, LB, LB), jnp.float32),
            pltpu.VMEM((2, 32, LB), jnp.float32),
            pltpu.SemaphoreType.DMA,
            pltpu.SemaphoreType.DMA,
        ],
        compiler_params=pltpu.CompilerParams(use_tc_tiling_on_sc=True,
                                             needs_layout_passes=False),
    )
    def embed(x_hbm, t_hbm, out_hbm, xbuf, idxbuf, basebuf, gbuf, ebuf,
              gsem, ssem):
        wid = lax.axis_index("s") * NC + lax.axis_index("c")
        pltpu.sync_copy(x_hbm.at[:, pl.ds(wid * LB, LB)], xbuf)

        lane = lax.iota(jnp.int32, 16)

        def compute_idx(s, p):
            for lc in range(8):
                r = xbuf[s, pl.ds(lc * 16, 16)]
                idxbuf[p, pl.ds(lc * 16, 16)] = r >> 2
                basebuf[p, pl.ds(lc * 16, 16)] = (r & 3) * 32

        def fire_gather(p):
            pltpu.async_copy(t_hbm.at[idxbuf.at[p]], gbuf.at[p], gsem)

        def wait_gather():
            pltpu.make_async_copy(
                t_hbm.at[idxbuf.at[0]], gbuf.at[0], gsem).wait()

        def extract(p):
            bases = [basebuf[p, pl.ds(lc * 16, 16)] for lc in range(8)]
            rows = [lane + (lc * 16) for lc in range(8)]
            for f in range(32):
                for lc in range(8):
                    vals = plsc.load_gather(
                        gbuf.at[p], [rows[lc], bases[lc] + f])
                    ebuf[p, f, pl.ds(lc * 16, 16)] = vals

        def fire_stores(s, p):
            for fi in range(4):
                pltpu.async_copy(
                    ebuf.at[p, pl.ds(fi * 8, 8)], out_hbm.at[s, fi, wid],
                    ssem)

        def wait_store():
            pltpu.make_async_copy(
                ebuf.at[0, pl.ds(0, 8)], out_hbm.at[0, 0, 0], ssem).wait()

        for p in range(2):
            compute_idx(p, p)
            fire_gather(p)

        @pl.loop(0, S, step=2)
        def _steps(g):
            for b in range(2):
                s = g + b
                wait_gather()

                @pl.when(s >= 2)
                def _drain():
                    for _ in range(4):
                        wait_store()

                extract(b)
                fire_stores(s, b)

                @pl.when(s + 2 < S)
                def _refill():
                    compute_idx(s + 2, b)
                    fire_gather(b)

        for _ in range(8):
            wait_store()

    return embed(x_t, table4)


def kernel(x, embedding):
    B, S = x.shape
    V, D = embedding.shape
    x_t = x.astype(jnp.int32).T
    table4 = embedding.reshape(V // GROUP, D * GROUP)
    out5 = _embed_call(S, V // GROUP, x_t, table4)
    out = out5.transpose(0, 1, 3, 2, 4).reshape(S, D, B)
    return out.transpose(2, 0, 1)


# speedup vs baseline: 1.3639x; 1.1814x over previous
import functools

import jax
import jax.numpy as jnp
from jax import lax
from jax.experimental import pallas as pl
from jax.experimental.pallas import tpu as pltpu
from jax.experimental.pallas import tpu_sc as plsc

NC, NS = 2, 16
NW = NC * NS
LB = 128
GROUP = 4


def _embed_call(S, V4, x_t, table4):
    mesh = plsc.VectorSubcoreMesh(core_axis_name="c", subcore_axis_name="s")

    @functools.partial(
        pl.kernel,
        out_type=jax.ShapeDtypeStruct((S, 4, NW, 8, LB), jnp.float32),
        mesh=mesh,
        scratch_types=[
            pltpu.VMEM((S, LB), jnp.int32),
            pltpu.VMEM((2, LB), jnp.int32),
            pltpu.VMEM((2, 8, 16), jnp.int32),
            pltpu.VMEM((2, LB, LB), jnp.float32),
            pltpu.VMEM((2, 32, LB), jnp.float32),
            pltpu.SemaphoreType.DMA,
            pltpu.SemaphoreType.DMA,
        ],
        compiler_params=pltpu.CompilerParams(use_tc_tiling_on_sc=True,
                                             needs_layout_passes=False),
    )
    def embed(x_hbm, t_hbm, out_hbm, xbuf, idxbuf, basebuf, gbuf, ebuf,
              gsem, ssem):
        wid = lax.axis_index("s") * NC + lax.axis_index("c")
        pltpu.sync_copy(x_hbm.at[:, pl.ds(wid * LB, LB)], xbuf)
        t4 = t_hbm

        lane = lax.iota(jnp.int32, 16)
        rot = [((lane + k) & 15) for k in range(16)]

        def compute_idx(s, p):
            for lc in range(8):
                r = xbuf[s, pl.ds(lc * 16, 16)]
                idxbuf[p, pl.ds(lc * 16, 16)] = r >> 2
                basebuf[p, lc, :] = (r & 3) * 32

        def fire_gather(p):
            pltpu.async_copy(t4.at[idxbuf.at[p]], gbuf.at[p], gsem)

        def wait_gather():
            pltpu.make_async_copy(
                t4.at[idxbuf.at[0]], gbuf.at[0], gsem).wait()

        def extract(p):
            @pl.loop(0, 8)
            def _blocks(lc):
                lvec = lane + lc * 16
                base = basebuf[p, lc, :]
                for fb in range(2):
                    bb = base + (fb * 16)
                    for k in range(16):
                        fvec = rot[k] + (fb * 16)
                        vals = plsc.load_gather(
                            gbuf.at[p], [lvec, bb + rot[k]])
                        plsc.store_scatter(ebuf.at[p], [fvec, lvec], vals)

        def fire_stores(s, p):
            for fi in range(4):
                pltpu.async_copy(
                    ebuf.at[p, pl.ds(fi * 8, 8)], out_hbm.at[s, fi, wid],
                    ssem)

        def wait_store():
            pltpu.make_async_copy(
                ebuf.at[0, pl.ds(0, 8)], out_hbm.at[0, 0, 0], ssem).wait()

        for p in range(2):
            compute_idx(p, p)
            fire_gather(p)

        @pl.loop(0, S, step=2)
        def _steps(g):
            for b in range(2):
                s = g + b
                wait_gather()

                @pl.when(s >= 2)
                def _drain():
                    for _ in range(4):
                        wait_store()

                extract(b)
                fire_stores(s, b)

                @pl.when(s + 2 < S)
                def _refill():
                    compute_idx(s + 2, b)
                    fire_gather(b)

        for _ in range(8):
            wait_store()

    return embed(x_t, table4)


def kernel(x, embedding):
    B, S = x.shape
    V, D = embedding.shape
    x_t = x.astype(jnp.int32).T
    table4 = embedding.reshape(V // GROUP, D * GROUP)
    out5 = _embed_call(S, V // GROUP, x_t, table4)
    out = out5.transpose(0, 1, 3, 2, 4).reshape(S, D, B)
    return out.transpose(2, 0, 1)


# speedup vs baseline: 2.2322x; 1.6366x over previous
import functools

import jax
import jax.numpy as jnp
from jax import lax
from jax.experimental import pallas as pl
from jax.experimental.pallas import tpu as pltpu
from jax.experimental.pallas import tpu_sc as plsc

NC, NS = 2, 16
NW = NC * NS
LB = 128
GROUP = 4


def _transpose_call(table_t):
    F, V = table_t.shape
    NCOLS = V // LB
    NITER = 246
    mesh = plsc.VectorSubcoreMesh(core_axis_name="c", subcore_axis_name="s")

    @functools.partial(
        pl.kernel,
        out_type=jax.ShapeDtypeStruct(((NCOLS + 1) * 32, LB), jnp.float32),
        mesh=mesh,
        scratch_types=[
            pltpu.VMEM((2, 32, LB), jnp.float32),
            pltpu.VMEM((2, 32, LB), jnp.float32),
            pltpu.SemaphoreType.DMA,
            pltpu.SemaphoreType.DMA,
        ],
        compiler_params=pltpu.CompilerParams(use_tc_tiling_on_sc=True,
                                             needs_layout_passes=False,
                                             disable_bounds_checks=True),
    )
    def transpose(t_hbm, stage_hbm, tin, tbuf, isem, osem):
        wid = lax.axis_index("s") * NC + lax.axis_index("c")
        lane = lax.iota(jnp.int32, 16)
        rot = [((lane + k) & 15) for k in range(16)]
        lane4x32 = (lane & 3) * 32
        laneq = lane >> 2

        def src_off(t):
            return pl.multiple_of(jnp.minimum(wid + 32 * t, NCOLS) * LB, LB)

        def fire_in(t, p):
            pltpu.async_copy(
                t_hbm.at[:, pl.ds(src_off(t), LB)], tin.at[p], isem)

        def wait_in():
            pltpu.make_async_copy(
                t_hbm.at[:, pl.ds(0, LB)], tin.at[0], isem).wait()

        def fire_out(t, p):
            pltpu.async_copy(
                tbuf.at[p],
                stage_hbm.at[pl.ds(pl.multiple_of(src_off(t) >> 2, 32), 32)],
                osem)

        def wait_out():
            pltpu.make_async_copy(
                tbuf.at[0], stage_hbm.at[pl.ds(0, 32)], osem).wait()

        def do_transpose(p):
            @pl.loop(0, 8)
            def _blocks(rb):
                rrv = lane + rb * 16
                qv = laneq + rb * 4
                for f0 in range(0, 32, 16):
                    for k in range(16):
                        fv = rot[k] + f0
                        vals = plsc.load_gather(tin.at[p], [fv, rrv])
                        plsc.store_scatter(
                            tbuf.at[p], [qv, lane4x32 + fv], vals)

        for p in range(2):
            fire_in(p, p)

        @pl.loop(0, NITER, step=2)
        def _cols(t0):
            for p in range(2):
                t = t0 + p
                wait_in()

                @pl.when(t >= 2)
                def _drain():
                    wait_out()

                do_transpose(p)
                fire_out(t, p)

                @pl.when(t + 2 < NITER)
                def _refill():
                    fire_in(t + 2, p)

        for _ in range(2):
            wait_out()

    return transpose(table_t)


def _embed_call(S, V4, x_t, table4):
    mesh = plsc.VectorSubcoreMesh(core_axis_name="c", subcore_axis_name="s")

    @functools.partial(
        pl.kernel,
        out_type=jax.ShapeDtypeStruct((S, 4, NW, 8, LB), jnp.float32),
        mesh=mesh,
        scratch_types=[
            pltpu.VMEM((S, LB), jnp.int32),
            pltpu.VMEM((2, LB), jnp.int32),
            pltpu.VMEM((2, 8, 16), jnp.int32),
            pltpu.VMEM((2, LB, LB), jnp.float32),
            pltpu.VMEM((2, 32, LB), jnp.float32),
            pltpu.SemaphoreType.DMA,
            pltpu.SemaphoreType.DMA,
        ],
        compiler_params=pltpu.CompilerParams(use_tc_tiling_on_sc=True,
                                             needs_layout_passes=False),
    )
    def embed(x_hbm, t_hbm, out_hbm, xbuf, idxbuf, basebuf, gbuf, ebuf,
              gsem, ssem):
        wid = lax.axis_index("s") * NC + lax.axis_index("c")
        pltpu.sync_copy(x_hbm.at[:, pl.ds(wid * LB, LB)], xbuf)
        t4 = t_hbm

        lane = lax.iota(jnp.int32, 16)
        rot = [((lane + k) & 15) for k in range(16)]

        def compute_idx(s, p):
            for lc in range(8):
                r = xbuf[s, pl.ds(lc * 16, 16)]
                idxbuf[p, pl.ds(lc * 16, 16)] = r >> 2
                basebuf[p, lc, :] = (r & 3) * 32

        def fire_gather(p):
            pltpu.async_copy(t4.at[idxbuf.at[p]], gbuf.at[p], gsem)

        def wait_gather():
            pltpu.make_async_copy(
                t4.at[idxbuf.at[0]], gbuf.at[0], gsem).wait()

        def extract(p):
            @pl.loop(0, 8)
            def _blocks(lc):
                lvec = lane + lc * 16
                base = basebuf[p, lc, :]
                for fb in range(2):
                    bb = base + (fb * 16)
                    for k in range(16):
                        fvec = rot[k] + (fb * 16)
                        vals = plsc.load_gather(
                            gbuf.at[p], [lvec, bb + rot[k]])
                        plsc.store_scatter(ebuf.at[p], [fvec, lvec], vals)

        def fire_stores(s, p):
            for fi in range(4):
                pltpu.async_copy(
                    ebuf.at[p, pl.ds(fi * 8, 8)], out_hbm.at[s, fi, wid],
                    ssem)

        def wait_store():
            pltpu.make_async_copy(
                ebuf.at[0, pl.ds(0, 8)], out_hbm.at[0, 0, 0], ssem).wait()

        for p in range(2):
            compute_idx(p, p)
            fire_gather(p)

        @pl.loop(0, S, step=2)
        def _steps(g):
            for b in range(2):
                s = g + b
                wait_gather()

                @pl.when(s >= 2)
                def _drain():
                    for _ in range(4):
                        wait_store()

                extract(b)
                fire_stores(s, b)

                @pl.when(s + 2 < S)
                def _refill():
                    compute_idx(s + 2, b)
                    fire_gather(b)

        for _ in range(8):
            wait_store()

    return embed(x_t, table4)


def kernel(x, embedding):
    B, S = x.shape
    V, D = embedding.shape
    x_t = x.astype(jnp.int32).T
    table4 = _transpose_call(embedding.T)
    out5 = _embed_call(S, table4.shape[0], x_t, table4)
    out = out5.transpose(0, 1, 3, 2, 4).reshape(S, D, B)
    return out.transpose(2, 0, 1)
